# baseline (speedup 1.0000x reference)
import functools
import math

import numpy as np
import jax
import jax.numpy as jnp
from jax import lax
from jax.experimental import pallas as pl
from jax.experimental.pallas import tpu as pltpu
from jax.experimental.pallas import tpu_sc as plsc

_D_MODEL = 512
_MAX_LEN = 2048
_SCALE = math.sqrt(float(_D_MODEL))
_NUM_CORES = 2
_NUM_SUBCORES = 16
_NUM_WORKERS = _NUM_CORES * _NUM_SUBCORES
_LANES = 16


def _sinusoidal_pe_np(max_len, d_model):
    pos = np.arange(max_len, dtype=np.float32)[:, None]
    div = np.exp(
        np.arange(0, d_model, 2, dtype=np.float32) * (-math.log(10000.0) / d_model)
    )
    pe = np.zeros((max_len, d_model), dtype=np.float32)
    pe[:, 0::2] = np.sin(pos * div)
    pe[:, 1::2] = np.cos(pos * div)
    return pe


_PE_NP = _sinusoidal_pe_np(_MAX_LEN, _D_MODEL)


@functools.lru_cache(maxsize=None)
def _build(n_rows, seq_len, d_model):
    per_w = n_rows // _NUM_WORKERS
    chunk = 64
    n_chunks = per_w // chunk
    mesh = plsc.VectorSubcoreMesh(core_axis_name="c", subcore_axis_name="s")

    def body(tok_hbm, table_hbm, pe_hbm, out_hbm, idx_v, rows_v, pe_v, sem):
        wid = lax.axis_index("s") * _NUM_CORES + lax.axis_index("c")

        def do_chunk(ci, carry):
            base = wid * per_w + ci * chunk
            t0 = lax.rem(base, seq_len)
            pltpu.sync_copy(tok_hbm.at[pl.ds(base, chunk)], idx_v)
            gather = pltpu.async_copy(table_hbm.at[idx_v], rows_v, sem)
            pltpu.sync_copy(pe_hbm.at[pl.ds(t0, chunk)], pe_v)
            gather.wait()

            def do_row(r, c2):
                for j in range(d_model // _LANES):
                    sl = pl.ds(j * _LANES, _LANES)
                    rows_v[r, sl] = rows_v[r, sl] * _SCALE + pe_v[r, sl]
                return c2

            lax.fori_loop(0, chunk, do_row, 0)
            pltpu.sync_copy(rows_v, out_hbm.at[pl.ds(base, chunk)])
            return carry

        lax.fori_loop(0, n_chunks, do_chunk, 0)

    return pl.kernel(
        body,
        out_type=jax.ShapeDtypeStruct((n_rows, d_model), jnp.float32),
        mesh=mesh,
        scratch_types=[
            pltpu.VMEM((chunk,), jnp.int32),
            pltpu.VMEM((chunk, d_model), jnp.float32),
            pltpu.VMEM((chunk, d_model), jnp.float32),
            pltpu.SemaphoreType.DMA,
        ],
    )


def kernel(tokens, table):
    b, t = tokens.shape
    v, d = table.shape
    tok_flat = tokens.reshape(-1).astype(jnp.int32)
    pe = jnp.asarray(_PE_NP[:t])
    out = _build(b * t, t, d)(tok_flat, table, pe)
    return out.reshape(b, t, d)


# speedup vs baseline: 1.1920x; 1.1920x over previous
import functools
import math

import numpy as np
import jax
import jax.numpy as jnp
from jax import lax
from jax.experimental import pallas as pl
from jax.experimental.pallas import tpu as pltpu
from jax.experimental.pallas import tpu_sc as plsc

_D_MODEL = 512
_MAX_LEN = 2048
_SCALE = math.sqrt(float(_D_MODEL))
_NUM_CORES = 2
_NUM_SUBCORES = 16
_NUM_WORKERS = _NUM_CORES * _NUM_SUBCORES
_LANES = 16
_CHUNK = 32
_RING = 3


def _sinusoidal_pe_np(max_len, d_model):
    pos = np.arange(max_len, dtype=np.float32)[:, None]
    div = np.exp(
        np.arange(0, d_model, 2, dtype=np.float32) * (-math.log(10000.0) / d_model)
    )
    pe = np.zeros((max_len, d_model), dtype=np.float32)
    pe[:, 0::2] = np.sin(pos * div)
    pe[:, 1::2] = np.cos(pos * div)
    return pe


_PE_NP = _sinusoidal_pe_np(_MAX_LEN, _D_MODEL)


@functools.lru_cache(maxsize=None)
def _build(n_rows, seq_len, d_model):
    per_w = n_rows // _NUM_WORKERS
    n_chunks = per_w // _CHUNK
    prime = min(_RING - 1, n_chunks)
    mesh = plsc.VectorSubcoreMesh(core_axis_name="c", subcore_axis_name="s")

    def body(tok_hbm, table_hbm, pe_hbm, out_hbm, *scr):
        idx = scr[0:3]
        gbuf = scr[3:6]
        pbuf = scr[6:9]
        gsem = scr[9:12]
        psem = scr[12:15]
        osem = scr[15:18]
        wid = lax.axis_index("s") * _NUM_CORES + lax.axis_index("c")

        in_flight = {}
        out_flight = {}

        def issue_in(ci):
            s = ci % _RING
            base = wid * per_w + ci * _CHUNK
            t0 = lax.rem(base, seq_len)
            pltpu.sync_copy(tok_hbm.at[pl.ds(base, _CHUNK)], idx[s])
            g = pltpu.async_copy(table_hbm.at[idx[s]], gbuf[s], gsem[s])
            p = pltpu.async_copy(pe_hbm.at[pl.ds(t0, _CHUNK)], pbuf[s], psem[s])
            in_flight[ci] = (g, p)

        for ci in range(prime):
            issue_in(ci)

        for ci in range(n_chunks):
            s = ci % _RING
            g, p = in_flight.pop(ci)
            g.wait()
            p.wait()

            def do_row(r, carry):
                for j in range(d_model // _LANES):
                    sl = pl.ds(j * _LANES, _LANES)
                    plsc.addupdate(pbuf[s].at[r, sl], gbuf[s][r, sl] * _SCALE)
                return carry

            lax.fori_loop(0, _CHUNK, do_row, 0)

            base = wid * per_w + ci * _CHUNK
            out_flight[s] = pltpu.async_copy(
                pbuf[s], out_hbm.at[pl.ds(base, _CHUNK)], osem[s]
            )

            nxt = ci + prime
            if nxt < n_chunks:
                ns = nxt % _RING
                if ns in out_flight:
                    out_flight.pop(ns).wait()
                issue_in(nxt)

        for d in out_flight.values():
            d.wait()

    return pl.kernel(
        body,
        out_type=jax.ShapeDtypeStruct((n_rows, d_model), jnp.float32),
        mesh=mesh,
        scratch_types=(
            [pltpu.VMEM((_CHUNK,), jnp.int32) for _ in range(_RING)]
            + [pltpu.VMEM((_CHUNK, d_model), jnp.float32) for _ in range(_RING)]
            + [pltpu.VMEM((_CHUNK, d_model), jnp.float32) for _ in range(_RING)]
            + [pltpu.SemaphoreType.DMA for _ in range(3 * _RING)]
        ),
    )


def kernel(tokens, table):
    b, t = tokens.shape
    v, d = table.shape
    tok_flat = tokens.reshape(-1).astype(jnp.int32)
    pe = jnp.asarray(_PE_NP[:t])
    out = _build(b * t, t, d)(tok_flat, table, pe)
    return out.reshape(b, t, d)
